# baseline (speedup 1.0000x reference)
import functools

import jax
import jax.numpy as jnp
from jax import lax
from jax.experimental import pallas as pl
from jax.experimental.pallas import tpu as pltpu
from jax.experimental.pallas import tpu_sc as plsc

K_CB = 8192
D_CB = 32
TB = 256


def _argmin_body(z_ref, cbt2_ref, insq_ref, cbsq_ref, idx_ref):
    fl = z_ref[...]
    cbt2 = cbt2_ref[...]
    zb = fl.astype(jnp.bfloat16)
    cbtb = cbt2.astype(jnp.bfloat16)
    mm2 = jax.lax.dot_general(zb, cbtb, (((1,), (0,)), ((), ())),
                              preferred_element_type=jnp.float32)
    insq = insq_ref[...]
    cbsq = cbsq_ref[...]
    dist = (cbsq[None, :] + insq[:, None]) + mm2

    def argmin_first(dhalf, offs):
        m = jnp.min(dhalf, axis=1)
        ii = lax.broadcasted_iota(jnp.int32, dhalf.shape, 1) + offs
        return m, jnp.min(jnp.where(dhalf == m[:, None], ii, K_CB), axis=1)

    vA, iA = argmin_first(dist[:, : K_CB // 2], 0)
    vB, iB = argmin_first(dist[:, K_CB // 2:], K_CB // 2)
    ai = vA.view(jnp.int32)
    bias = ((ai >> 16) & 1) + 0x7FFF
    vA16 = ((ai + bias) & ~jnp.int32(0xFFFF)).view(jnp.float32)
    pickA = vA16 <= vB
    idx_ref[...] = jnp.where(pickA, iA, iB)


def _vq_argmin(flat, cbt2, insq, cbsq):
    bn = flat.shape[0]
    return pl.pallas_call(
        _argmin_body,
        grid=(bn // TB,),
        in_specs=[
            pl.BlockSpec((TB, D_CB), lambda i: (i, 0)),
            pl.BlockSpec((D_CB, K_CB), lambda i: (0, 0)),
            pl.BlockSpec((TB,), lambda i: (i,)),
            pl.BlockSpec((K_CB,), lambda i: (0,)),
        ],
        out_specs=pl.BlockSpec((TB,), lambda i: (i,)),
        out_shape=jax.ShapeDtypeStruct((bn,), jnp.int32),
    )(flat, cbt2, insq, cbsq)


def _gather_codes(cb128, idx_flat):
    info = plsc.get_sparse_core_info()
    nc, ns = info.num_cores, info.num_subcores
    nw = nc * ns
    bn = idx_flat.shape[0]
    bpw = bn // nw
    ch = 128
    nch = bpw // ch
    idx2 = idx_flat.reshape(nw, bpw)
    mesh = plsc.VectorSubcoreMesh(core_axis_name="c", subcore_axis_name="s")

    @functools.partial(
        pl.kernel, mesh=mesh,
        out_type=jax.ShapeDtypeStruct((bn, 128), jnp.float32),
        scratch_types=[
            pltpu.VMEM((bpw,), jnp.int32),
            pltpu.VMEM((bpw, 128), jnp.float32),
            pltpu.SemaphoreType.DMA,
        ],
    )
    def gather(cb_hbm, idx_hbm, out_hbm, idx_v, rows_v, sem):
        wid = lax.axis_index("s") * nc + lax.axis_index("c")
        pltpu.sync_copy(idx_hbm.at[wid], idx_v)
        copies = [
            pltpu.async_copy(cb_hbm.at[idx_v.at[pl.ds(j * ch, ch)]],
                             rows_v.at[pl.ds(j * ch, ch)], sem)
            for j in range(nch)
        ]
        for c in copies:
            c.wait()
        pltpu.sync_copy(rows_v, out_hbm.at[pl.ds(wid * bpw, bpw)])

    return gather(cb128, idx2)


def kernel(z_e_x, codebook):
    b, n, c = z_e_x.shape
    flat = z_e_x.reshape(b * n, c)
    insq = jnp.sum(z_e_x * z_e_x, axis=2).reshape(-1)
    cbsq = jnp.sum(codebook * codebook, axis=1)
    cbt2 = codebook.T + codebook.T
    idx_flat = _vq_argmin(flat, cbt2, insq, cbsq)
    cb128 = jnp.pad(codebook, ((0, 0), (0, 128 - c)))
    codes = _gather_codes(cb128, idx_flat)[:, :c].reshape(b, n, c)
    return (codes, codes, idx_flat.reshape(b, n))


# speedup vs baseline: 1.0630x; 1.0630x over previous
import functools

import jax
import jax.numpy as jnp
from jax import lax
from jax.experimental import pallas as pl
from jax.experimental.pallas import tpu as pltpu
from jax.experimental.pallas import tpu_sc as plsc

K_CB = 8192
D_CB = 32
TB = 256


def _argmin_body(z_ref, cbt2_ref, insq_ref, cbsq_ref, idx_ref):
    fl = z_ref[...]
    cbt = cbt2_ref[...]
    zb = fl.astype(jnp.bfloat16)
    cbtb = cbt.astype(jnp.bfloat16)
    mm = jax.lax.dot_general(zb, cbtb, (((1,), (0,)), ((), ())),
                             preferred_element_type=jnp.float32)
    insq = insq_ref[...]
    cbsq = cbsq_ref[...]
    dist = (cbsq[None, :] + insq[:, None]) + 2.0 * mm

    def argmin_first(dhalf, offs):
        m = jnp.min(dhalf, axis=1)
        ii = lax.broadcasted_iota(jnp.int32, dhalf.shape, 1) + offs
        return m, jnp.min(jnp.where(dhalf == m[:, None], ii, K_CB), axis=1)

    vA, iA = argmin_first(dist[:, : K_CB // 2], 0)
    vB, iB = argmin_first(dist[:, K_CB // 2:], K_CB // 2)
    ai = vA.view(jnp.int32)
    bias = ((ai >> 16) & 1) + 0x7FFF
    vA16 = ((ai + bias) & ~jnp.int32(0xFFFF)).view(jnp.float32)
    pickA = vA16 <= vB
    idx_ref[...] = jnp.where(pickA, iA, iB)


def _vq_argmin(flat, cbt2, insq, cbsq):
    bn = flat.shape[0]
    return pl.pallas_call(
        _argmin_body,
        grid=(bn // TB,),
        in_specs=[
            pl.BlockSpec((TB, D_CB), lambda i: (i, 0)),
            pl.BlockSpec((D_CB, K_CB), lambda i: (0, 0)),
            pl.BlockSpec((TB,), lambda i: (i,)),
            pl.BlockSpec((K_CB,), lambda i: (0,)),
        ],
        out_specs=pl.BlockSpec((TB,), lambda i: (i,)),
        out_shape=jax.ShapeDtypeStruct((bn,), jnp.int32),
    )(flat, cbt2, insq, cbsq)


def _gather_codes(cb128, idx_flat):
    info = plsc.get_sparse_core_info()
    nc, ns = info.num_cores, info.num_subcores
    nw = nc * ns
    bn = idx_flat.shape[0]
    bpw = bn // nw
    ch = 128
    nch = bpw // ch
    idx2 = idx_flat.reshape(nw, bpw)
    mesh = plsc.VectorSubcoreMesh(core_axis_name="c", subcore_axis_name="s")

    @functools.partial(
        pl.kernel, mesh=mesh,
        out_type=jax.ShapeDtypeStruct((bn, 128), jnp.float32),
        scratch_types=[
            pltpu.VMEM((bpw,), jnp.int32),
            pltpu.VMEM((bpw, 128), jnp.float32),
            pltpu.SemaphoreType.DMA,
        ],
    )
    def gather(cb_hbm, idx_hbm, out_hbm, idx_v, rows_v, sem):
        wid = lax.axis_index("s") * nc + lax.axis_index("c")
        pltpu.sync_copy(idx_hbm.at[wid], idx_v)
        copies = [
            pltpu.async_copy(cb_hbm.at[idx_v.at[pl.ds(j * ch, ch)]],
                             rows_v.at[pl.ds(j * ch, ch)], sem)
            for j in range(nch)
        ]
        for c in copies:
            c.wait()
        pltpu.sync_copy(rows_v, out_hbm.at[pl.ds(wid * bpw, bpw)])

    return gather(cb128, idx2)


def kernel(z_e_x, codebook):
    b, n, c = z_e_x.shape
    flat = z_e_x.reshape(b * n, c)
    insq = jnp.sum(z_e_x * z_e_x, axis=2).reshape(-1)
    cbsq = jnp.sum(codebook * codebook, axis=1)
    cbt = codebook.T
    idx_flat = _vq_argmin(flat, cbt, insq, cbsq)
    cb128 = jnp.pad(codebook, ((0, 0), (0, 128 - c)))
    codes = _gather_codes(cb128, idx_flat)[:, :c].reshape(b, n, c)
    return (codes, codes, idx_flat.reshape(b, n))
